# Initial kernel scaffold; baseline (speedup 1.0000x reference)
import functools

import jax
import jax.numpy as jnp
from jax import lax
from jax.experimental import pallas as pl
from jax.experimental.pallas import tpu as pltpu
from jax.experimental.pallas import tpu_sc as plsc

B = 16384
D = 64
K = 20

_info = plsc.get_sparse_core_info()
NC, NS = _info.num_cores, _info.num_subcores
NW = NC * NS
BPW = B // NW
C = 64
NCHUNK = BPW // C
NSEG = (C * K) // 128


def _sc_scores(centers, contexts, neg2d, input_emb, output_emb):
    mesh = plsc.VectorSubcoreMesh(core_axis_name="c", subcore_axis_name="s")

    @functools.partial(
        pl.kernel,
        mesh=mesh,
        out_type=jax.ShapeDtypeStruct((NW, 1 + K, BPW), jnp.float32),
        scratch_types=[
            pltpu.VMEM((C,), jnp.int32),
            pltpu.VMEM((C,), jnp.int32),
            pltpu.VMEM((NSEG, 128), jnp.int32),
            pltpu.VMEM((C, D), jnp.float32),
            pltpu.VMEM((C, D), jnp.float32),
            pltpu.VMEM((C * K, D), jnp.float32),
            pltpu.VMEM((1 + K, BPW), jnp.float32),
            pltpu.SemaphoreType.DMA,
        ],
    )
    def body(cen_h, ctx_h, neg_h, iemb_h, oemb_h, out_h,
             cidx, xidx, nidx, cbuf, xbuf, nbuf, sbuf, sem):
        wid = lax.axis_index("s") * NC + lax.axis_index("c")
        base = wid * BPW
        lanes = lax.iota(jnp.int32, 16)

        def chunk(t, _):
            off = t * C
            start = base + off
            pltpu.sync_copy(cen_h.at[pl.ds(start, C)], cidx)
            pltpu.sync_copy(ctx_h.at[pl.ds(start, C)], xidx)
            pltpu.sync_copy(neg_h.at[pl.ds(start * K // 128, NSEG)], nidx)
            copies = [
                pltpu.make_async_copy(iemb_h.at[cidx], cbuf, sem),
                pltpu.make_async_copy(oemb_h.at[xidx], xbuf, sem),
            ]
            for j in range(NSEG):
                copies.append(pltpu.make_async_copy(
                    oemb_h.at[nidx.at[j]], nbuf.at[pl.ds(j * 128, 128)], sem))
            for cp in copies:
                cp.start()
            for cp in copies:
                cp.wait()
            for g in range(C // 16):
                rows = g * 16 + lanes
                rows_k = rows * K

                def dstep(dd, accs):
                    col = jnp.broadcast_to(dd, (16,)).astype(jnp.int32)
                    ccol = plsc.load_gather(cbuf, [rows, col])
                    xcol = plsc.load_gather(xbuf, [rows, col])
                    new = [accs[0] + ccol * xcol]
                    for k in range(K):
                        ncol = plsc.load_gather(nbuf, [rows_k + k, col])
                        new.append(accs[k + 1] + ccol * ncol)
                    return tuple(new)

                accs = lax.fori_loop(
                    0, D, dstep,
                    tuple(jnp.zeros((16,), jnp.float32) for _ in range(1 + K)))
                sl = pl.ds(off + g * 16, 16)
                sbuf[0, sl] = -accs[0]
                for k in range(K):
                    sbuf[k + 1, sl] = accs[k + 1]
            return 0

        lax.fori_loop(0, NCHUNK, chunk, 0)
        pltpu.sync_copy(sbuf, out_h.at[wid])

    return body(centers, contexts, neg2d, input_emb, output_emb)


def _loss_tc(scores):
    def body(s_ref, o_ref):
        x = s_ref[...]
        sp = jnp.maximum(x, 0.0) + jnp.log1p(jnp.exp(-jnp.abs(x)))
        o_ref[0, 0] = jnp.sum(sp) * (1.0 / B)

    return pl.pallas_call(
        body,
        out_shape=jax.ShapeDtypeStruct((1, 1), jnp.float32),
        out_specs=pl.BlockSpec(memory_space=pltpu.SMEM),
    )(scores)


def kernel(input_emb, output_emb, centers, contexts, negatives):
    neg2d = negatives.astype(jnp.int32).reshape(B * K // 128, 128)
    scores = _sc_scores(centers.astype(jnp.int32), contexts.astype(jnp.int32),
                        neg2d, input_emb, output_emb)
    loss = _loss_tc(scores.reshape(NW * (1 + K), BPW))
    return loss[0, 0]


# speedup vs baseline: 3.9254x; 3.9254x over previous
import functools

import jax
import jax.numpy as jnp
from jax import lax
from jax.experimental import pallas as pl
from jax.experimental.pallas import tpu as pltpu
from jax.experimental.pallas import tpu_sc as plsc

B = 16384
D = 64
K = 20

_info = plsc.get_sparse_core_info()
NC, NS = _info.num_cores, _info.num_subcores
NW = NC * NS
BPW = B // NW
C = 32
NCHUNK = BPW // C
NSEG = (C * K) // 128
SB = (1 + K) * BPW


def _sc_scores(centers, contexts, neg_flat, iemb2, oemb2):
    mesh = plsc.VectorSubcoreMesh(core_axis_name="c", subcore_axis_name="s")

    @functools.partial(
        pl.kernel,
        mesh=mesh,
        out_type=jax.ShapeDtypeStruct((NW * SB,), jnp.float32),
        compiler_params=pltpu.CompilerParams(needs_layout_passes=False),
        scratch_types=[
            pltpu.VMEM((C,), jnp.int32),
            pltpu.VMEM((C,), jnp.int32),
            pltpu.VMEM((C * K,), jnp.int32),
            pltpu.VMEM((C,), jnp.int32),
            pltpu.VMEM((C,), jnp.int32),
            pltpu.VMEM((C * K,), jnp.int32),
            pltpu.VMEM((C, 2 * D), jnp.float32),
            pltpu.VMEM((C, 2 * D), jnp.float32),
            pltpu.VMEM((C * K, 2 * D), jnp.float32),
            pltpu.VMEM((SB,), jnp.float32),
            pltpu.SemaphoreType.DMA,
        ],
    )
    def body(cen_h, ctx_h, neg_h, iemb_h, oemb_h, out_h,
             cidx, xidx, nidx, cidx2, xidx2, nidx2, cbuf, xbuf, nbuf,
             sbuf, sem):
        wid = lax.axis_index("s") * NC + lax.axis_index("c")
        base = wid * BPW
        lanes = lax.iota(jnp.int32, 16)

        def chunk(t, _):
            off = t * C
            start = base + off
            pltpu.sync_copy(cen_h.at[pl.ds(start, C)], cidx)
            pltpu.sync_copy(ctx_h.at[pl.ds(start, C)], xidx)
            pltpu.sync_copy(neg_h.at[pl.ds(start * K, C * K)], nidx)
            for v in range(C // 16):
                sl = pl.ds(v * 16, 16)
                cidx2[sl] = cidx[sl] >> 1
                xidx2[sl] = xidx[sl] >> 1
            for v in range((C * K) // 16):
                sl = pl.ds(v * 16, 16)
                nidx2[sl] = nidx[sl] >> 1
            copies = [
                pltpu.make_async_copy(iemb_h.at[cidx2], cbuf, sem),
                pltpu.make_async_copy(oemb_h.at[xidx2], xbuf, sem),
            ]
            for j in range(NSEG):
                copies.append(pltpu.make_async_copy(
                    oemb_h.at[nidx2.at[pl.ds(j * 128, 128)]],
                    nbuf.at[pl.ds(j * 128, 128)], sem))
            for cp in copies:
                cp.start()
            for cp in copies:
                cp.wait()
            for g in range(C // 16):
                rows = g * 16 + lanes
                rows_k = rows * K
                cpar = (cidx[pl.ds(g * 16, 16)] & 1) * D
                xpar = (xidx[pl.ds(g * 16, 16)] & 1) * D
                npars = [(plsc.load_gather(nidx, [rows_k + k]) & 1) * D
                         for k in range(K)]

                def dstep(dd, accs):
                    col = jnp.broadcast_to(dd, (16,)).astype(jnp.int32)
                    ccol = plsc.load_gather(cbuf, [rows, col + cpar])
                    xcol = plsc.load_gather(xbuf, [rows, col + xpar])
                    new = [accs[0] + ccol * xcol]
                    for k in range(K):
                        ncol = plsc.load_gather(nbuf, [rows_k + k, col + npars[k]])
                        new.append(accs[k + 1] + ccol * ncol)
                    return tuple(new)

                accs = lax.fori_loop(
                    0, D, dstep,
                    tuple(jnp.zeros((16,), jnp.float32) for _ in range(1 + K)))
                eoff = off + g * 16
                sbuf[pl.ds(eoff, 16)] = -accs[0]
                for k in range(K):
                    sbuf[pl.ds((k + 1) * BPW + eoff, 16)] = accs[k + 1]
            return 0

        lax.fori_loop(0, NCHUNK, chunk, 0)
        pltpu.sync_copy(sbuf, out_h.at[pl.ds(wid * SB, SB)])

    return body(centers, contexts, neg_flat, iemb2, oemb2)


def _loss_tc(scores):
    def body(s_ref, o_ref):
        x = s_ref[...]
        sp = jnp.maximum(x, 0.0) + jnp.log1p(jnp.exp(-jnp.abs(x)))
        o_ref[0, 0] = jnp.sum(sp) * (1.0 / B)

    return pl.pallas_call(
        body,
        out_shape=jax.ShapeDtypeStruct((1, 1), jnp.float32),
        out_specs=pl.BlockSpec(memory_space=pltpu.SMEM),
    )(scores)


def kernel(input_emb, output_emb, centers, contexts, negatives):
    iemb2 = input_emb.reshape(-1, 2 * D)
    oemb2 = output_emb.reshape(-1, 2 * D)
    neg_flat = negatives.astype(jnp.int32).reshape(B * K)
    scores = _sc_scores(centers.astype(jnp.int32), contexts.astype(jnp.int32),
                        neg_flat, iemb2, oemb2)
    loss = _loss_tc(scores.reshape(NW * (1 + K), BPW))
    return loss[0, 0]


# speedup vs baseline: 4.2245x; 1.0762x over previous
import functools

import jax
import jax.numpy as jnp
from jax import lax
from jax.experimental import pallas as pl
from jax.experimental.pallas import tpu as pltpu
from jax.experimental.pallas import tpu_sc as plsc

B = 16384
D = 64
K = 20
V = 1000000

_info = plsc.get_sparse_core_info()
NC, NS = _info.num_cores, _info.num_subcores
NW = NC * NS
BPW = B // NW
C = 32
NCHUNK = BPW // C
NSEG = (C * K) // 128
SB = (1 + K) * BPW

CB = 2048
NBLK = (V + CB - 1) // CB
VP = NBLK * CB


def _tp_body(t_ref, o_ref):
    blk = t_ref[...]
    o_ref[...] = jnp.concatenate(
        [blk.T, jnp.zeros((CB, D), jnp.float32)], axis=1)


def _transpose_tc(embT):
    return pl.pallas_call(
        _tp_body,
        grid=(NBLK,),
        in_specs=[pl.BlockSpec((D, CB), lambda i: (0, i))],
        out_specs=pl.BlockSpec((CB, 2 * D), lambda i: (i, 0)),
        out_shape=jax.ShapeDtypeStruct((VP, 2 * D), jnp.float32),
    )(embT)


def _sc_scores(centers, contexts, neg_flat, iemb, oemb):
    mesh = plsc.VectorSubcoreMesh(core_axis_name="c", subcore_axis_name="s")

    @functools.partial(
        pl.kernel,
        mesh=mesh,
        out_type=jax.ShapeDtypeStruct((NW * SB,), jnp.float32),
        compiler_params=pltpu.CompilerParams(needs_layout_passes=False),
        scratch_types=[
            pltpu.VMEM((C,), jnp.int32),
            pltpu.VMEM((C,), jnp.int32),
            pltpu.VMEM((C * K,), jnp.int32),
            pltpu.VMEM((C, 2 * D), jnp.float32),
            pltpu.VMEM((C, 2 * D), jnp.float32),
            pltpu.VMEM((C * K, 2 * D), jnp.float32),
            pltpu.VMEM((SB,), jnp.float32),
            pltpu.SemaphoreType.DMA,
        ],
    )
    def body(cen_h, ctx_h, neg_h, iemb_h, oemb_h, out_h,
             cidx, xidx, nidx, cbuf, xbuf, nbuf, sbuf, sem):
        wid = lax.axis_index("s") * NC + lax.axis_index("c")
        base = wid * BPW
        lanes = lax.iota(jnp.int32, 16)

        def chunk(t, _):
            off = t * C
            start = base + off
            pltpu.sync_copy(cen_h.at[pl.ds(start, C)], cidx)
            pltpu.sync_copy(ctx_h.at[pl.ds(start, C)], xidx)
            pltpu.sync_copy(neg_h.at[pl.ds(start * K, C * K)], nidx)
            copies = [
                pltpu.make_async_copy(iemb_h.at[cidx], cbuf, sem),
                pltpu.make_async_copy(oemb_h.at[xidx], xbuf, sem),
            ]
            for j in range(NSEG):
                copies.append(pltpu.make_async_copy(
                    oemb_h.at[nidx.at[pl.ds(j * 128, 128)]],
                    nbuf.at[pl.ds(j * 128, 128)], sem))
            for cp in copies:
                cp.start()
            for cp in copies:
                cp.wait()
            for g in range(C // 16):
                rows = g * 16 + lanes
                rows_k = rows * K

                def dstep(dd, accs):
                    col = jnp.broadcast_to(dd, (16,)).astype(jnp.int32)
                    ccol = plsc.load_gather(cbuf, [rows, col])
                    xcol = plsc.load_gather(xbuf, [rows, col])
                    new = [accs[0] + ccol * xcol]
                    for k in range(K):
                        ncol = plsc.load_gather(nbuf, [rows_k + k, col])
                        new.append(accs[k + 1] + ccol * ncol)
                    return tuple(new)

                accs = lax.fori_loop(
                    0, D, dstep,
                    tuple(jnp.zeros((16,), jnp.float32) for _ in range(1 + K)))
                eoff = off + g * 16
                sbuf[pl.ds(eoff, 16)] = -accs[0]
                for k in range(K):
                    sbuf[pl.ds((k + 1) * BPW + eoff, 16)] = accs[k + 1]
            return 0

        lax.fori_loop(0, NCHUNK, chunk, 0)
        pltpu.sync_copy(sbuf, out_h.at[pl.ds(wid * SB, SB)])

    return body(centers, contexts, neg_flat, iemb, oemb)


def _loss_tc(scores):
    def body(s_ref, o_ref):
        x = s_ref[...]
        sp = jnp.maximum(x, 0.0) + jnp.log1p(jnp.exp(-jnp.abs(x)))
        o_ref[0, 0] = jnp.sum(sp) * (1.0 / B)

    return pl.pallas_call(
        body,
        out_shape=jax.ShapeDtypeStruct((1, 1), jnp.float32),
        out_specs=pl.BlockSpec(memory_space=pltpu.SMEM),
    )(scores)


def kernel(input_emb, output_emb, centers, contexts, negatives):
    iemb = _transpose_tc(input_emb.T)
    oemb = _transpose_tc(output_emb.T)
    neg_flat = negatives.astype(jnp.int32).reshape(B * K)
    scores = _sc_scores(centers.astype(jnp.int32), contexts.astype(jnp.int32),
                        neg_flat, iemb, oemb)
    loss = _loss_tc(scores.reshape(NW * (1 + K), BPW))
    return loss[0, 0]


# speedup vs baseline: 5.7330x; 1.3571x over previous
import functools

import jax
import jax.numpy as jnp
from jax import lax
from jax.experimental import pallas as pl
from jax.experimental.pallas import tpu as pltpu
from jax.experimental.pallas import tpu_sc as plsc

B = 16384
D = 64
K = 20
V = 1000000

_info = plsc.get_sparse_core_info()
NC, NS = _info.num_cores, _info.num_subcores
NW = NC * NS
BPW = B // NW
C = 16
NCHUNK = BPW // C
SB = (1 + K) * BPW

CB = 2048
NPAIR = (V + 2 * CB - 1) // (2 * CB)
VP = NPAIR * 2 * CB


def _tp_body(a_ref, b_ref, o_ref):
    o_ref[...] = jnp.concatenate([a_ref[...].T, b_ref[...].T], axis=1)


def _transpose_tc(embT):
    return pl.pallas_call(
        _tp_body,
        grid=(NPAIR,),
        in_specs=[pl.BlockSpec((D, CB), lambda i: (0, 2 * i)),
                  pl.BlockSpec(
                      (D, CB),
                      lambda i: (0, jnp.minimum(2 * i + 1, (V - 1) // CB)))],
        out_specs=pl.BlockSpec((CB, 2 * D), lambda i: (i, 0)),
        out_shape=jax.ShapeDtypeStruct((NPAIR * CB, 2 * D), jnp.float32),
    )(embT, embT)


def _sc_scores(centers, contexts, neg_flat, iemb, oemb):
    mesh = plsc.VectorSubcoreMesh(core_axis_name="c", subcore_axis_name="s")

    @functools.partial(
        pl.kernel,
        mesh=mesh,
        out_type=jax.ShapeDtypeStruct((NW * SB,), jnp.float32),
        compiler_params=pltpu.CompilerParams(needs_layout_passes=False),
        scratch_types=[
            pltpu.VMEM((BPW,), jnp.int32),
            pltpu.VMEM((BPW,), jnp.int32),
            pltpu.VMEM((BPW * K,), jnp.int32),
            pltpu.VMEM((BPW,), jnp.int32),
            pltpu.VMEM((BPW,), jnp.int32),
            pltpu.VMEM((BPW * K,), jnp.int32),
            pltpu.VMEM((C, 2 * D), jnp.float32),
            pltpu.VMEM((C, 2 * D), jnp.float32),
            pltpu.VMEM((C, 2 * D), jnp.float32),
            pltpu.VMEM((C, 2 * D), jnp.float32),
            pltpu.VMEM((C * K, 2 * D), jnp.float32),
            pltpu.VMEM((C * K, 2 * D), jnp.float32),
            pltpu.VMEM((SB,), jnp.float32),
            pltpu.SemaphoreType.DMA,
            pltpu.SemaphoreType.DMA,
        ],
    )
    def body(cen_h, ctx_h, neg_h, iemb_h, oemb_h, out_h,
             cidxa, xidxa, nidxa, cidx2, xidx2, nidx2,
             cb0, cb1, xb0, xb1, nb0, nb1, sbuf, sem0, sem1):
        wid = lax.axis_index("s") * NC + lax.axis_index("c")
        base = wid * BPW
        lanes = lax.iota(jnp.int32, 16)
        pltpu.sync_copy(cen_h.at[pl.ds(base, BPW)], cidxa)
        pltpu.sync_copy(ctx_h.at[pl.ds(base, BPW)], xidxa)
        pltpu.sync_copy(neg_h.at[pl.ds(base * K, BPW * K)], nidxa)

        def remap(src, dst, n):
            def step(i, _):
                sl = pl.ds(i * 16, 16)
                ids = src[sl]
                dst[sl] = ((ids >> 12) << 11) + (ids & 2047)
                return 0
            lax.fori_loop(0, n // 16, step, 0)

        remap(cidxa, cidx2, BPW)
        remap(xidxa, xidx2, BPW)
        remap(nidxa, nidx2, BPW * K)
        bufs = ((cb0, xb0, nb0, sem0), (cb1, xb1, nb1, sem1))

        def descs(t, p):
            cb, xb, nb, sem = bufs[p]
            off = t * C
            return [
                pltpu.make_async_copy(
                    iemb_h.at[cidx2.at[pl.ds(off, C)]], cb, sem),
                pltpu.make_async_copy(
                    oemb_h.at[xidx2.at[pl.ds(off, C)]], xb, sem),
                pltpu.make_async_copy(
                    oemb_h.at[nidx2.at[pl.ds(off * K, C * K)]], nb, sem),
            ]

        def compute(t, p):
            cb, xb, nb, _ = bufs[p]
            rows_k = lanes * K
            off = t * C
            cpar = ((cidxa[pl.ds(off, 16)] >> 11) & 1) * D
            xpar = ((xidxa[pl.ds(off, 16)] >> 11) & 1) * D
            npars = [
                ((plsc.load_gather(nidxa, [off * K + rows_k + k]) >> 11) & 1)
                * D for k in range(K)]

            def dstep(dd, accs):
                new = list(accs)
                for u in range(4):
                    col = jnp.broadcast_to(
                        dd * 4 + u, (16,)).astype(jnp.int32)
                    ccol = plsc.load_gather(cb, [lanes, col + cpar])
                    xcol = plsc.load_gather(xb, [lanes, col + xpar])
                    new[0] = new[0] + ccol * xcol
                    for k in range(K):
                        ncol = plsc.load_gather(
                            nb, [rows_k + k, col + npars[k]])
                        new[k + 1] = new[k + 1] + ccol * ncol
                return tuple(new)

            accs = lax.fori_loop(
                0, D // 4, dstep,
                tuple(jnp.zeros((16,), jnp.float32) for _ in range(1 + K)))
            eoff = t * C
            sbuf[pl.ds(eoff, 16)] = -accs[0]
            for k in range(K):
                sbuf[pl.ds((k + 1) * BPW + eoff, 16)] = accs[k + 1]

        for cp in descs(0, 0):
            cp.start()

        def pair(tt, _):
            for p in range(2):
                t = tt * 2 + p

                @pl.when(t + 1 < NCHUNK)
                def _():
                    for cp in descs(t + 1, 1 - p):
                        cp.start()

                for cp in descs(t, p):
                    cp.wait()
                compute(t, p)
            return 0

        lax.fori_loop(0, NCHUNK // 2, pair, 0)
        pltpu.sync_copy(sbuf, out_h.at[pl.ds(wid * SB, SB)])

    return body(centers, contexts, neg_flat, iemb, oemb)


def _loss_tc(scores):
    def body(s_ref, o_ref):
        x = s_ref[...]
        sp = jnp.maximum(x, 0.0) + jnp.log1p(jnp.exp(-jnp.abs(x)))
        o_ref[0, 0] = jnp.sum(sp) * (1.0 / B)

    return pl.pallas_call(
        body,
        out_shape=jax.ShapeDtypeStruct((1, 1), jnp.float32),
        out_specs=pl.BlockSpec(memory_space=pltpu.SMEM),
    )(scores)


def kernel(input_emb, output_emb, centers, contexts, negatives):
    iemb = _transpose_tc(input_emb.T)
    oemb = _transpose_tc(output_emb.T)
    neg_flat = negatives.astype(jnp.int32).reshape(B * K)
    scores = _sc_scores(centers.astype(jnp.int32), contexts.astype(jnp.int32),
                        neg_flat, iemb, oemb)
    loss = _loss_tc(scores.reshape(NW * (1 + K), BPW))
    return loss[0, 0]
